# Initial kernel scaffold; baseline (speedup 1.0000x reference)
import jax
import jax.numpy as jnp
from jax.experimental import pallas as pl


def kernel(x, codebook):
    raise NotImplementedError("write your pallas kernel here")


# speedup vs baseline: 1.8332x; 1.8332x over previous
import jax
import jax.numpy as jnp
from jax.experimental import pallas as pl

_B, _C, _H, _W = 8, 64, 128, 128
_K = 512
_N = _H * _W
_BLK = 2048
_STEPS = _B * _N // _BLK


def _vq_body(x_ref, cb_ref, xrec_ref, idx_ref, acc_ref):
    i = pl.program_id(0)
    xb = x_ref[0]
    cb = cb_ref[...]

    mm = jax.lax.dot(cb, xb, precision=jax.lax.Precision.DEFAULT,
                     preferred_element_type=jnp.float32)
    xsq = jnp.sum(xb * xb, axis=0, keepdims=True)
    e2 = jnp.sum(cb * cb, axis=1, keepdims=True)
    d = (xsq - 2.0 * mm) + e2

    m = jnp.min(d, axis=0, keepdims=True)
    kiota = jax.lax.broadcasted_iota(jnp.int32, d.shape, 0)
    kid = jnp.min(jnp.where(d == m, kiota, _K), axis=0, keepdims=True)
    idx_ref[0] = kid

    onehot = (kiota == kid).astype(jnp.float32)
    zq = jax.lax.dot_general(cb, onehot, (((0,), (0,)), ((), ())),
                             precision=jax.lax.Precision.HIGHEST,
                             preferred_element_type=jnp.float32)

    xrec_ref[0] = xb + (zq - xb)
    part = jnp.sum((zq - xb) ** 2, axis=0, keepdims=True)

    @pl.when(i == 0)
    def _init():
        acc_ref[0] = part

    @pl.when(i > 0)
    def _accum():
        acc_ref[0] = acc_ref[0] + part


def kernel(x, codebook):
    xq = x.reshape(_B, _C, _N)
    xrec3, idx3, acc = pl.pallas_call(
        _vq_body,
        grid=(_STEPS,),
        in_specs=[
            pl.BlockSpec((1, _C, _BLK), lambda i: (i // (_N // _BLK), 0, i % (_N // _BLK))),
            pl.BlockSpec((_K, _C), lambda i: (0, 0)),
        ],
        out_specs=[
            pl.BlockSpec((1, _C, _BLK), lambda i: (i // (_N // _BLK), 0, i % (_N // _BLK))),
            pl.BlockSpec((1, 1, _BLK), lambda i: (i, 0, 0)),
            pl.BlockSpec((1, 1, _BLK), lambda i: (0, 0, 0)),
        ],
        out_shape=[
            jax.ShapeDtypeStruct((_B, _C, _N), jnp.float32),
            jax.ShapeDtypeStruct((_STEPS, 1, _BLK), jnp.int32),
            jax.ShapeDtypeStruct((1, 1, _BLK), jnp.float32),
        ],
    )(xq, codebook)

    x_rec = xrec3.reshape(_B, _C, _H, _W)
    indices = idx3.reshape(_B, _H, _W)
    loss = 1.25 * jnp.sum(acc) / (_B * _C * _H * _W)
    return x_rec, x_rec, indices, loss


# speedup vs baseline: 2.7532x; 1.5019x over previous
import jax
import jax.numpy as jnp
from jax.experimental import pallas as pl
from jax.experimental.pallas import tpu as pltpu

_B, _C, _H, _W = 8, 64, 128, 128
_K = 512
_N = _H * _W
_BLK = 2048
_STEPS = _B * _N // _BLK


def _vq_body(x_ref, cb_ref, xrec_ref, idx_ref, acc_ref, kio_ref):
    i = pl.program_id(0)
    xb = x_ref[0]
    cb = cb_ref[...]

    @pl.when(i == 0)
    def _fill_iota():
        kio_ref[...] = jax.lax.broadcasted_iota(
            jnp.int32, (_K, _BLK), 0).astype(jnp.float32)

    mm = jax.lax.dot(cb, xb, precision=jax.lax.Precision.DEFAULT,
                     preferred_element_type=jnp.float32)
    xsq = jnp.sum(xb * xb, axis=0, keepdims=True)
    e2 = jnp.sum(cb * cb, axis=1, keepdims=True)
    d = (xsq - 2.0 * mm) + e2

    m = jnp.min(d, axis=0, keepdims=True)
    kio = kio_ref[...]
    kidf = jnp.min(jnp.where(d == m, kio, float(_K)), axis=0, keepdims=True)
    idx_ref[0] = kidf.astype(jnp.int32)

    onehot = (kio == kidf).astype(jnp.bfloat16)
    cbh = cb.astype(jnp.bfloat16)
    cbl = (cb - cbh.astype(jnp.float32)).astype(jnp.bfloat16)
    dims = (((0,), (0,)), ((), ()))
    zq = (jax.lax.dot_general(cbh, onehot, dims, preferred_element_type=jnp.float32)
          + jax.lax.dot_general(cbl, onehot, dims, preferred_element_type=jnp.float32))

    xrec_ref[0] = xb + (zq - xb)
    part = jnp.sum((zq - xb) ** 2, axis=0, keepdims=True)

    @pl.when(i == 0)
    def _init():
        acc_ref[0] = part

    @pl.when(i > 0)
    def _accum():
        acc_ref[0] = acc_ref[0] + part


def kernel(x, codebook):
    xq = x.reshape(_B, _C, _N)
    xrec3, idx3, acc = pl.pallas_call(
        _vq_body,
        grid=(_STEPS,),
        in_specs=[
            pl.BlockSpec((1, _C, _BLK), lambda i: (i // (_N // _BLK), 0, i % (_N // _BLK))),
            pl.BlockSpec((_K, _C), lambda i: (0, 0)),
        ],
        out_specs=[
            pl.BlockSpec((1, _C, _BLK), lambda i: (i // (_N // _BLK), 0, i % (_N // _BLK))),
            pl.BlockSpec((1, 1, _BLK), lambda i: (i, 0, 0)),
            pl.BlockSpec((1, 1, _BLK), lambda i: (0, 0, 0)),
        ],
        out_shape=[
            jax.ShapeDtypeStruct((_B, _C, _N), jnp.float32),
            jax.ShapeDtypeStruct((_STEPS, 1, _BLK), jnp.int32),
            jax.ShapeDtypeStruct((1, 1, _BLK), jnp.float32),
        ],
        scratch_shapes=[pltpu.VMEM((_K, _BLK), jnp.float32)],
    )(xq, codebook)

    x_rec = xrec3.reshape(_B, _C, _H, _W)
    indices = idx3.reshape(_B, _H, _W)
    loss = 1.25 * jnp.sum(acc) / (_B * _C * _H * _W)
    return x_rec, x_rec, indices, loss


# speedup vs baseline: 3.7364x; 1.3571x over previous
import jax
import jax.numpy as jnp
from jax.experimental import pallas as pl
from jax.experimental.pallas import tpu as pltpu

_B, _C, _H, _W = 8, 64, 128, 128
_K = 512
_N = _H * _W
_RB = 16
_BLK = _RB * _W
_NB = _H // _RB
_STEPS = _B * _NB


def _vq_body(x_ref, cb_ref, kio_ref, xrec_ref, idx_ref, acc_ref):
    i = pl.program_id(0)
    xb = x_ref[0].reshape(_C, _BLK)
    cb = cb_ref[...]
    kio = kio_ref[...]

    mm = jax.lax.dot(cb, xb, precision=jax.lax.Precision.DEFAULT,
                     preferred_element_type=jnp.float32)
    xsq = jnp.sum(xb * xb, axis=0, keepdims=True)
    e2 = jnp.sum(cb * cb, axis=1, keepdims=True)
    d = (xsq - 2.0 * mm) + e2

    m = jnp.min(d, axis=0, keepdims=True)
    kidf = jnp.min(jnp.where(d == m, kio, float(_K)), axis=0, keepdims=True)
    idx_ref[0] = kidf.astype(jnp.int32).reshape(_RB, _W)

    onehot = (kio == kidf).astype(jnp.bfloat16)
    cbh = cb.astype(jnp.bfloat16)
    cbl = (cb - cbh.astype(jnp.float32)).astype(jnp.bfloat16)
    dims = (((0,), (0,)), ((), ()))
    zq = (jax.lax.dot_general(cbh, onehot, dims, preferred_element_type=jnp.float32)
          + jax.lax.dot_general(cbl, onehot, dims, preferred_element_type=jnp.float32))

    xrec_ref[0] = (xb + (zq - xb)).reshape(_C, _RB, _W)
    part = jnp.sum((zq - xb) ** 2, axis=0, keepdims=True)

    @pl.when(i == 0)
    def _init():
        acc_ref[0] = part

    @pl.when(i > 0)
    def _accum():
        acc_ref[0] = acc_ref[0] + part


def kernel(x, codebook):
    kio = jax.lax.broadcasted_iota(jnp.int32, (_K, _BLK), 0).astype(jnp.float32)
    xrec, idx, acc = pl.pallas_call(
        _vq_body,
        grid=(_STEPS,),
        in_specs=[
            pl.BlockSpec((1, _C, _RB, _W), lambda i: (i // _NB, 0, i % _NB, 0)),
            pl.BlockSpec((_K, _C), lambda i: (0, 0)),
            pl.BlockSpec((_K, _BLK), lambda i: (0, 0)),
        ],
        out_specs=[
            pl.BlockSpec((1, _C, _RB, _W), lambda i: (i // _NB, 0, i % _NB, 0)),
            pl.BlockSpec((1, _RB, _W), lambda i: (i // _NB, i % _NB, 0)),
            pl.BlockSpec((1, 1, _BLK), lambda i: (0, 0, 0)),
        ],
        out_shape=[
            jax.ShapeDtypeStruct((_B, _C, _H, _W), jnp.float32),
            jax.ShapeDtypeStruct((_B, _H, _W), jnp.int32),
            jax.ShapeDtypeStruct((1, 1, _BLK), jnp.float32),
        ],
    )(x, codebook, kio)

    loss = 1.25 * jnp.sum(acc) / (_B * _C * _H * _W)
    return xrec, xrec, idx, loss
